# baseline (speedup 1.0000x reference)
import functools

import jax
import jax.numpy as jnp
from jax.experimental import pallas as pl
from jax.experimental.pallas import tpu as pltpu

_RADIUS = 0.015
_IT = 512
_JT = 8192
_XR = 72


def _fused_body(xa_ref, ipt_ref, opt_ref, out_ref, acc_ref):
    j = pl.program_id(1)
    nj = pl.num_programs(1)

    @pl.when(j == 0)
    def _init():
        acc_ref[...] = jnp.zeros_like(acc_ref)

    opb = opt_ref[...]
    ipt = ipt_ref[...]
    no2 = jnp.sum(opb * opb, axis=1, keepdims=True)
    ni2 = jnp.sum(ipt * ipt, axis=0, keepdims=True)
    so = jnp.sum(opb, axis=1, keepdims=True)
    si = jnp.sum(ipt, axis=0, keepdims=True)

    opl = (2.0 * opb).astype(jnp.bfloat16)
    ipl = ipt.astype(jnp.bfloat16)
    dot2 = jax.lax.dot_general(
        opl, ipl, (((1,), (0,)), ((), ())),
        preferred_element_type=jnp.float32)

    d2 = (no2 + ni2) - dot2
    eterm = (no2 - 2.0 * so) + (ni2 + 2.0 * si + 3.0)
    w = jnp.where(d2 < _RADIUS * _RADIUS, jax.lax.rsqrt(eterm - dot2), 0.0)

    xa = xa_ref[...]
    acc_ref[...] += jax.lax.dot_general(
        xa, w, (((1,), (1,)), ((), ())),
        preferred_element_type=jnp.float32)

    @pl.when(j == nj - 1)
    def _finalize():
        acc = acc_ref[...]
        num = acc[:64, :]
        dem = acc[64:65, :]
        dem = jnp.where(dem > 0.0, dem, 1.0)
        out_ref[...] = num / dem


@jax.jit
def kernel(x, inp_positions, out_positions):
    b, n_in = x.shape
    n_out = out_positions.shape[0]
    perm_in = jnp.argsort(inp_positions[:, 0])
    perm_out = jnp.argsort(out_positions[:, 0])
    inv_out = jnp.argsort(perm_out)
    x = x[:, perm_in]
    inp_positions = inp_positions[perm_in]
    out_positions = out_positions[perm_out]
    xa = jnp.concatenate(
        [x,
         jnp.ones((1, n_in), dtype=x.dtype),
         jnp.zeros((_XR - b - 1, n_in), dtype=x.dtype)], axis=0)
    ipt = jnp.concatenate(
        [inp_positions.T, jnp.zeros((5, n_in), inp_positions.dtype)], axis=0)
    opb = jnp.concatenate(
        [out_positions, jnp.zeros((n_out, 5), out_positions.dtype)], axis=1)

    grid = (n_out // _IT, n_in // _JT)
    out = pl.pallas_call(
        _fused_body,
        grid=grid,
        in_specs=[
            pl.BlockSpec((_XR, _JT), lambda i, j: (0, j)),
            pl.BlockSpec((8, _JT), lambda i, j: (0, j)),
            pl.BlockSpec((_IT, 8), lambda i, j: (i, 0)),
        ],
        out_specs=pl.BlockSpec((b, _IT), lambda i, j: (0, i)),
        out_shape=jax.ShapeDtypeStruct((b, n_out), x.dtype),
        scratch_shapes=[pltpu.VMEM((_XR, _IT), jnp.float32)],
        compiler_params=pltpu.CompilerParams(
            dimension_semantics=("arbitrary", "arbitrary")),
    )(xa, ipt, opb)
    return out[:, inv_out]


# speedup vs baseline: 1.9384x; 1.9384x over previous
import functools

import jax
import jax.numpy as jnp
from jax.experimental import pallas as pl
from jax.experimental.pallas import tpu as pltpu

_RADIUS = 0.015
_IT = 512
_JT = 8192
_XR = 72


def _fused_body(xa_ref, ipt_ref, opt_ref, out_ref, acc_ref):
    j = pl.program_id(1)
    nj = pl.num_programs(1)

    @pl.when(j == 0)
    def _init():
        acc_ref[...] = jnp.zeros_like(acc_ref)

    opb = opt_ref[...]
    ipt = ipt_ref[...]
    no2 = jnp.sum(opb * opb, axis=1, keepdims=True)
    ni2 = jnp.sum(ipt * ipt, axis=0, keepdims=True)
    so = jnp.sum(opb, axis=1, keepdims=True)
    si = jnp.sum(ipt, axis=0, keepdims=True)

    opl = (2.0 * opb).astype(jnp.bfloat16)
    ipl = ipt.astype(jnp.bfloat16)
    dot2 = jax.lax.dot_general(
        opl, ipl, (((1,), (0,)), ((), ())),
        preferred_element_type=jnp.float32)

    d2 = (no2 + ni2) - dot2
    ao = (no2 - 2.0 * so).astype(jnp.bfloat16)
    bi = (ni2 + 2.0 * si + 3.0).astype(jnp.bfloat16)
    ysq = (ao + bi) - dot2.astype(jnp.bfloat16)
    w = jnp.where(d2 < _RADIUS * _RADIUS, jax.lax.rsqrt(ysq),
                  jnp.bfloat16(0.0))

    xa = xa_ref[...]
    acc_ref[...] += jax.lax.dot_general(
        xa, w, (((1,), (1,)), ((), ())),
        preferred_element_type=jnp.float32)

    @pl.when(j == nj - 1)
    def _finalize():
        acc = acc_ref[...]
        num = acc[:64, :]
        dem = acc[64:65, :]
        dem = jnp.where(dem > 0.0, dem, 1.0)
        out_ref[...] = num / dem


@jax.jit
def kernel(x, inp_positions, out_positions):
    b, n_in = x.shape
    n_out = out_positions.shape[0]
    xa = jnp.concatenate(
        [x,
         jnp.ones((1, n_in), dtype=x.dtype),
         jnp.zeros((_XR - b - 1, n_in), dtype=x.dtype)],
        axis=0).astype(jnp.bfloat16)
    ipt = jnp.concatenate(
        [inp_positions.T, jnp.zeros((5, n_in), inp_positions.dtype)], axis=0)
    opb = jnp.concatenate(
        [out_positions, jnp.zeros((n_out, 5), out_positions.dtype)], axis=1)

    grid = (n_out // _IT, n_in // _JT)
    out = pl.pallas_call(
        _fused_body,
        grid=grid,
        in_specs=[
            pl.BlockSpec((_XR, _JT), lambda i, j: (0, j)),
            pl.BlockSpec((8, _JT), lambda i, j: (0, j)),
            pl.BlockSpec((_IT, 8), lambda i, j: (i, 0)),
        ],
        out_specs=pl.BlockSpec((b, _IT), lambda i, j: (0, i)),
        out_shape=jax.ShapeDtypeStruct((b, n_out), x.dtype),
        scratch_shapes=[pltpu.VMEM((_XR, _IT), jnp.float32)],
        compiler_params=pltpu.CompilerParams(
            dimension_semantics=("arbitrary", "arbitrary")),
    )(xa, ipt, opb)
    return out
